# baseline (speedup 1.0000x reference)
import jax
import jax.numpy as jnp
from jax.experimental import pallas as pl
from jax.experimental.pallas import tpu as pltpu

H_BLK = 512
MC = 256


def _ffn_body(x_ref, w1_ref, b1_ref, w2_ref, b2_ref, o_ref, xb_ref):
    h = pl.program_id(1)
    cap = x_ref.shape[1]

    @pl.when(h == 0)
    def _():
        xb_ref[...] = x_ref[0].astype(jnp.bfloat16)

    w1 = w1_ref[0].astype(jnp.bfloat16)
    w2 = w2_ref[0].astype(jnp.bfloat16)
    b1 = b1_ref[0]
    b2 = b2_ref[0]

    for mi in range(cap // MC):
        sl = pl.ds(mi * MC, MC)
        xm = xb_ref[sl, :]
        y = jax.lax.dot_general(
            xm, w1, (((1,), (1,)), ((), ())),
            preferred_element_type=jnp.float32)
        y = jnp.maximum(y + b1, 0.0).astype(jnp.bfloat16)
        acc = jax.lax.dot_general(
            y, w2, (((1,), (0,)), ((), ())),
            preferred_element_type=jnp.float32)

        @pl.when(h == 0)
        def _():
            o_ref[0, sl, :] = acc + b2

        @pl.when(h != 0)
        def _():
            o_ref[0, sl, :] += acc


def kernel(x, fc1_w, fc1_b, fc2_w, fc2_b):
    E, CAP, D = x.shape
    H = fc1_w.shape[1]
    b1r = fc1_b.reshape(E, 1, H)
    b2r = fc2_b.reshape(E, 1, D)
    return pl.pallas_call(
        _ffn_body,
        grid=(E, H // H_BLK),
        in_specs=[
            pl.BlockSpec((1, CAP, D), lambda e, h: (e, 0, 0)),
            pl.BlockSpec((1, H_BLK, D), lambda e, h: (e, h, 0)),
            pl.BlockSpec((1, 1, H_BLK), lambda e, h: (e, 0, h)),
            pl.BlockSpec((1, H_BLK, D), lambda e, h: (e, h, 0)),
            pl.BlockSpec((1, 1, D), lambda e, h: (e, 0, 0)),
        ],
        out_specs=pl.BlockSpec((1, CAP, D), lambda e, h: (e, 0, 0)),
        out_shape=jax.ShapeDtypeStruct((E, CAP, D), jnp.float32),
        scratch_shapes=[pltpu.VMEM((CAP, D), jnp.bfloat16)],
        compiler_params=pltpu.CompilerParams(
            dimension_semantics=("parallel", "arbitrary"),
        ),
        name="fused_expert_ffn",
    )(x, fc1_w, b1r, fc2_w, b2r)


# speedup vs baseline: 1.3301x; 1.3301x over previous
import jax
import jax.numpy as jnp
from jax.experimental import pallas as pl
from jax.experimental.pallas import tpu as pltpu

H_BLK = 512
MC = 512


def _ffn_body(x_ref, w1_ref, b1_ref, w2_ref, b2_ref, o_ref, xb_ref):
    h = pl.program_id(1)
    cap = x_ref.shape[1]

    @pl.when(h == 0)
    def _():
        xb_ref[...] = x_ref[0].astype(jnp.bfloat16)

    w1 = w1_ref[0].astype(jnp.bfloat16)
    w2 = w2_ref[0].astype(jnp.bfloat16)
    b1 = b1_ref[0]
    b2 = b2_ref[0]

    for mi in range(cap // MC):
        sl = pl.ds(mi * MC, MC)
        xm = xb_ref[sl, :]
        y = jax.lax.dot_general(
            xm, w1, (((1,), (1,)), ((), ())),
            preferred_element_type=jnp.float32)
        y = jnp.maximum(y + b1, 0.0).astype(jnp.bfloat16)
        acc = jax.lax.dot_general(
            y, w2, (((1,), (0,)), ((), ())),
            preferred_element_type=jnp.float32)

        @pl.when(h == 0)
        def _():
            o_ref[0, sl, :] = acc + b2

        @pl.when(h != 0)
        def _():
            o_ref[0, sl, :] += acc


def kernel(x, fc1_w, fc1_b, fc2_w, fc2_b):
    E, CAP, D = x.shape
    H = fc1_w.shape[1]
    b1r = fc1_b.reshape(E, 1, H)
    b2r = fc2_b.reshape(E, 1, D)
    return pl.pallas_call(
        _ffn_body,
        grid=(E, H // H_BLK),
        in_specs=[
            pl.BlockSpec((1, CAP, D), lambda e, h: (e, 0, 0)),
            pl.BlockSpec((1, H_BLK, D), lambda e, h: (e, h, 0)),
            pl.BlockSpec((1, 1, H_BLK), lambda e, h: (e, 0, h)),
            pl.BlockSpec((1, H_BLK, D), lambda e, h: (e, h, 0)),
            pl.BlockSpec((1, 1, D), lambda e, h: (e, 0, 0)),
        ],
        out_specs=pl.BlockSpec((1, CAP, D), lambda e, h: (e, 0, 0)),
        out_shape=jax.ShapeDtypeStruct((E, CAP, D), jnp.float32),
        scratch_shapes=[pltpu.VMEM((CAP, D), jnp.bfloat16)],
        compiler_params=pltpu.CompilerParams(
            dimension_semantics=("parallel", "arbitrary"),
        ),
        name="fused_expert_ffn",
    )(x, fc1_w, b1r, fc2_w, b2r)


# speedup vs baseline: 1.7147x; 1.2892x over previous
import jax
import jax.numpy as jnp
from jax.experimental import pallas as pl
from jax.experimental.pallas import tpu as pltpu

H_BLK = 1024
MC = 2048


def _ffn_body(x_ref, w1_ref, b1_ref, w2_ref, b2_ref, o_ref, xb_ref):
    h = pl.program_id(1)
    cap = x_ref.shape[1]

    @pl.when(h == 0)
    def _():
        xb_ref[...] = x_ref[0].astype(jnp.bfloat16)

    w1 = w1_ref[0].astype(jnp.bfloat16)
    w2 = w2_ref[0].astype(jnp.bfloat16)
    b1 = b1_ref[0]
    b2 = b2_ref[0]

    for mi in range(cap // MC):
        sl = pl.ds(mi * MC, MC)
        xm = xb_ref[sl, :]
        y = jax.lax.dot_general(
            xm, w1, (((1,), (1,)), ((), ())),
            preferred_element_type=jnp.float32)
        y = jnp.maximum(y + b1, 0.0).astype(jnp.bfloat16)
        acc = jax.lax.dot_general(
            y, w2, (((1,), (0,)), ((), ())),
            preferred_element_type=jnp.float32)

        @pl.when(h == 0)
        def _():
            o_ref[0, sl, :] = acc + b2

        @pl.when(h != 0)
        def _():
            o_ref[0, sl, :] += acc


def kernel(x, fc1_w, fc1_b, fc2_w, fc2_b):
    E, CAP, D = x.shape
    H = fc1_w.shape[1]
    b1r = fc1_b.reshape(E, 1, H)
    b2r = fc2_b.reshape(E, 1, D)
    return pl.pallas_call(
        _ffn_body,
        grid=(E, H // H_BLK),
        in_specs=[
            pl.BlockSpec((1, CAP, D), lambda e, h: (e, 0, 0)),
            pl.BlockSpec((1, H_BLK, D), lambda e, h: (e, h, 0)),
            pl.BlockSpec((1, 1, H_BLK), lambda e, h: (e, 0, h)),
            pl.BlockSpec((1, H_BLK, D), lambda e, h: (e, h, 0)),
            pl.BlockSpec((1, 1, D), lambda e, h: (e, 0, 0)),
        ],
        out_specs=pl.BlockSpec((1, CAP, D), lambda e, h: (e, 0, 0)),
        out_shape=jax.ShapeDtypeStruct((E, CAP, D), jnp.float32),
        scratch_shapes=[pltpu.VMEM((CAP, D), jnp.bfloat16)],
        compiler_params=pltpu.CompilerParams(
            dimension_semantics=("parallel", "arbitrary"),
            vmem_limit_bytes=100 * 1024 * 1024,
        ),
        name="fused_expert_ffn",
    )(x, fc1_w, b1r, fc2_w, b2r)
